# baseline (speedup 1.0000x reference)
import jax
import jax.numpy as jnp
from jax.experimental import pallas as pl
from jax.experimental.pallas import tpu as pltpu

_OUT_FEATURES = 4096
_SIZE = 26
_LEVELS = 100
_LPAD = 128
_LOW = 0.0
_HIGH = 1.0
_BTILE = 256


def _encode_kernel(x_ref, pos_ref, val_ref, out_ref, w_ref):
    @pl.when(pl.program_id(0) == 0)
    def _build_w():
        vs = 1 - 2 * val_ref[...].astype(jnp.int32)
        vs_pad = jnp.concatenate(
            [vs, jnp.zeros((_LPAD - _LEVELS, _OUT_FEATURES), jnp.int32)], axis=0
        )
        ps = 1 - 2 * pos_ref[...].astype(jnp.int32)
        for s in range(_SIZE):
            w_ref[s * _LPAD : (s + 1) * _LPAD, :] = (ps[s : s + 1, :] * vs_pad).astype(jnp.int8)

    x = x_ref[...]
    idx = jnp.clip(
        jnp.round((x - _LOW) / (_HIGH - _LOW) * (_LEVELS - 1)), 0, _LEVELS - 1
    ).astype(jnp.int32)
    lanes = jax.lax.broadcasted_iota(jnp.int32, (x.shape[0], _LPAD), 1)
    m = jnp.concatenate(
        [(idx[:, s : s + 1] == lanes) for s in range(_SIZE)], axis=1
    ).astype(jnp.int8)
    t = jnp.dot(m, w_ref[...], preferred_element_type=jnp.int32)
    out_ref[...] = (t <= 0).astype(jnp.uint8)


def kernel(x, position_weight, value_weight):
    batch = x.shape[0]
    n_b = batch // _BTILE
    return pl.pallas_call(
        _encode_kernel,
        grid=(n_b,),
        in_specs=[
            pl.BlockSpec((_BTILE, _SIZE), lambda i: (i, 0)),
            pl.BlockSpec((_SIZE, _OUT_FEATURES), lambda i: (0, 0)),
            pl.BlockSpec((_LEVELS, _OUT_FEATURES), lambda i: (0, 0)),
        ],
        out_specs=pl.BlockSpec((_BTILE, _OUT_FEATURES), lambda i: (i, 0)),
        out_shape=jax.ShapeDtypeStruct((batch, _OUT_FEATURES), jnp.uint8),
        scratch_shapes=[pltpu.VMEM((_SIZE * _LPAD, _OUT_FEATURES), jnp.int8)],
    )(x, position_weight, value_weight)


# speedup vs baseline: 1.0150x; 1.0150x over previous
import jax
import jax.numpy as jnp
from jax.experimental import pallas as pl
from jax.experimental.pallas import tpu as pltpu

_OUT_FEATURES = 4096
_SIZE = 26
_LEVELS = 100
_LPAD = 128
_LOW = 0.0
_HIGH = 1.0
_BTILE = 512


def _encode_kernel(x_ref, pos_ref, val_ref, out_ref, w_ref):
    @pl.when(pl.program_id(0) == 0)
    def _build_w():
        vs = (1 - 2 * val_ref[...].astype(jnp.int32)).astype(jnp.bfloat16)
        vs_pad = jnp.concatenate(
            [vs, jnp.zeros((_LPAD - _LEVELS, _OUT_FEATURES), jnp.bfloat16)], axis=0
        )
        ps = (1 - 2 * pos_ref[...].astype(jnp.int32)).astype(jnp.bfloat16)
        for s in range(_SIZE):
            w_ref[s * _LPAD : (s + 1) * _LPAD, :] = ps[s : s + 1, :] * vs_pad

    x = x_ref[...]
    idx = jnp.clip(
        jnp.round((x - _LOW) / (_HIGH - _LOW) * (_LEVELS - 1)), 0, _LEVELS - 1
    ).astype(jnp.int32)
    lanes = jax.lax.broadcasted_iota(jnp.int32, (x.shape[0], _LPAD), 1)
    m = jnp.concatenate(
        [(idx[:, s : s + 1] == lanes) for s in range(_SIZE)], axis=1
    ).astype(jnp.bfloat16)
    t = jnp.dot(m, w_ref[...], preferred_element_type=jnp.float32)
    out_ref[...] = (t <= 0.0).astype(jnp.uint8)


def kernel(x, position_weight, value_weight):
    batch = x.shape[0]
    n_b = batch // _BTILE
    return pl.pallas_call(
        _encode_kernel,
        grid=(n_b,),
        in_specs=[
            pl.BlockSpec((_BTILE, _SIZE), lambda i: (i, 0)),
            pl.BlockSpec((_SIZE, _OUT_FEATURES), lambda i: (0, 0)),
            pl.BlockSpec((_LEVELS, _OUT_FEATURES), lambda i: (0, 0)),
        ],
        out_specs=pl.BlockSpec((_BTILE, _OUT_FEATURES), lambda i: (i, 0)),
        out_shape=jax.ShapeDtypeStruct((batch, _OUT_FEATURES), jnp.uint8),
        scratch_shapes=[pltpu.VMEM((_SIZE * _LPAD, _OUT_FEATURES), jnp.bfloat16)],
    )(x, position_weight, value_weight)
